# baseline (speedup 1.0000x reference)
import jax
import jax.numpy as jnp
from jax import lax
from jax.experimental import pallas as pl
from jax.experimental.pallas import tpu as pltpu
from jax.experimental.pallas import tpu_sc as plsc

_N = 1048576
_K = max(1, int(_N * 0.2))
_NT = 16
_L = 16
_NB = 256
_NR = 4
_SHIFTS = (24, 16, 8, 0)
_MASKHI = (0, -(1 << 24), -(1 << 16), -(1 << 8))
_MIN32 = -(2**31)


def _make(n, k):
    nt, l, nb, nr = _NT, _L, _NB, _NR
    c = n // nt
    nchunk = c // l
    tw = min(c, 8192)
    nw = c // tw
    twc = tw // l
    cap = min(c, 32768)

    mesh = plsc.VectorSubcoreMesh(
        core_axis_name="c", subcore_axis_name="s", num_cores=1, num_subcores=nt
    )

    def body(pred_hbm, tgt_hbm, out_hbm, xbuf, tbuf, hist, hist2, hl,
             totb, cgeb, cb, cbuf, shh, shc, sem_a, sem_b, sem_t0, sem_t1):
        wid = lax.axis_index("s")
        base = wid * c
        half = c // 2
        cp0 = pltpu.async_copy(pred_hbm.at[pl.ds(base, half)],
                               xbuf.at[pl.ds(0, half)], sem_a)
        cp1 = pltpu.async_copy(pred_hbm.at[pl.ds(base + half, half)],
                               xbuf.at[pl.ds(half, half)], sem_b)
        tcps = [None] * nw
        tcps[0] = pltpu.async_copy(tgt_hbm.at[pl.ds(base, tw)],
                                   tbuf.at[0], sem_t0)

        ones = jnp.full((l,), 1, jnp.int32)
        zeros = jnp.full((l,), 0, jnp.int32)
        min32 = jnp.int32(_MIN32)
        lane = lax.iota(jnp.int32, l)

        prefix = jnp.int32(0)
        krem = jnp.int32(k)
        d_last = jnp.int32(0)

        for r in range(nr):
            shift = _SHIFTS[r]
            maskhi = jnp.int32(_MASKHI[r])
            shift_v = jnp.full((l,), shift, jnp.int32)

            def zero_body(j):
                for s in range(l):
                    hist2[j * l + s, pl.ds(0, l)] = zeros

            plsc.parallel_loop(0, nb // l, unroll=2)(zero_body)

            prefix_c = prefix

            def hist_body(i, carry, maskhi=maskhi, shift_v=shift_v,
                          prefix_c=prefix_c):
                cnt = carry
                sl = pl.ds(i * l, l)
                ub = xbuf[sl]
                match = (ub & maskhi) == prefix_c
                digit = lax.shift_right_logical(ub, shift_v) & 0xFF
                plsc.addupdate_scatter(hist2, [digit, lane], ones,
                                       mask=match)
                safe = jnp.minimum(cnt, jnp.int32(cap))
                plsc.store_compressed(cbuf.at[pl.ds(safe, l)], ub,
                                      mask=match)
                return cnt + plsc.all_reduce_population_count(match)[0]

            if r == 0:
                def r0_body(i, maskhi=maskhi, shift_v=shift_v,
                            prefix_c=prefix_c):
                    sl = pl.ds(i * l, l)
                    b = xbuf[sl]
                    ub = jnp.where(b >= 0, b ^ min32, ~b)
                    xbuf[sl] = ub
                    match = (ub & maskhi) == prefix_c
                    digit = lax.shift_right_logical(ub, shift_v) & 0xFF
                    plsc.addupdate_scatter(hist2, [digit, lane], ones,
                                           mask=match)

                cp0.wait()
                plsc.parallel_loop(0, nchunk // 2, unroll=8)(r0_body)
                cp1.wait()
                plsc.parallel_loop(nchunk // 2, nchunk, unroll=8)(r0_body)
            elif r == 1:
                cnt = plsc.parallel_loop(0, nchunk, unroll=8,
                                         carry=jnp.int32(0))(hist_body)
                lcnt = jnp.minimum(cnt, jnp.int32(cap))
                ovf = cnt > jnp.int32(cap)
                cb[0, pl.ds(0, l)] = jnp.where(ovf, ones, zeros)
                pltpu.sync_copy(cb.at[0], shc.at[wid])
            else:
                def compact_scan(maskhi=maskhi, shift_v=shift_v,
                                 prefix_c=prefix_c, lcnt=lcnt):
                    def cbody(i, _):
                        for u in range(2):
                            off = (i * 2 + u) * l
                            ub = cbuf[pl.ds(off, l)]
                            valid = (off + lane) < lcnt
                            match = valid & ((ub & maskhi) == prefix_c)
                            digit = (lax.shift_right_logical(ub, shift_v)
                                     & 0xFF)
                            plsc.addupdate_scatter(hist2, [digit, lane],
                                                   ones, mask=match)
                        return 0

                    lax.fori_loop(0, (lcnt + 2 * l - 1) // (2 * l), cbody, 0)

                def full_scan(maskhi=maskhi, shift_v=shift_v,
                              prefix_c=prefix_c):
                    def fbody(i, maskhi=maskhi, shift_v=shift_v,
                              prefix_c=prefix_c):
                        ub = xbuf[pl.ds(i * l, l)]
                        match = (ub & maskhi) == prefix_c
                        digit = lax.shift_right_logical(ub, shift_v) & 0xFF
                        plsc.addupdate_scatter(hist2, [digit, lane], ones,
                                               mask=match)

                    plsc.parallel_loop(0, nchunk, unroll=4)(fbody)

                lax.cond(use_compact, compact_scan, full_scan)

            def fold_body(j):
                rows = j * l + lane
                acc = zeros
                for s in range(l):
                    cols = (lane + s) & (l - 1)
                    acc = acc + plsc.load_gather(hist2, [rows, cols])
                hist[pl.ds(j * l, l)] = acc

            plsc.parallel_loop(0, nb // l, unroll=2)(fold_body)

            pltpu.sync_copy(hist, shh.at[r, wid])
            plsc.subcore_barrier()
            pltpu.sync_copy(shh.at[r], hl)

            if r == 1:
                pltpu.sync_copy(shc, cb)
                oacc = cb[0, pl.ds(0, l)]
                for s in range(1, nt):
                    oacc = oacc + cb[s, pl.ds(0, l)]
                use_compact = jnp.sum(oacc) == 0

            def red_body(cc):
                sl = pl.ds(cc * l, l)
                acc = hl[0, sl]
                for s in range(1, nt):
                    acc = acc + hl[s, sl]
                totb[sl] = acc

            plsc.parallel_loop(0, nb // l, unroll=2)(red_body)

            krem_c = krem

            def scan_body(j, carry, krem_c=krem_c):
                acc, nfound = carry
                cc = (nb // l - 1) - j
                sl = pl.ds(cc * l, l)
                t16 = totb[sl]
                csum = plsc.cumsum(t16)
                ctot = jnp.sum(t16)
                cge = (acc + ctot) - csum + t16
                cgeb[sl] = cge
                nfound = nfound + jnp.sum(jnp.where(cge >= krem_c, ones, zeros))
                return (acc + ctot, nfound)

            _, nfound = lax.fori_loop(
                0, nb // l, scan_body, (jnp.int32(0), jnp.int32(0))
            )
            d = nfound - 1
            dvec = jnp.full((l,), d, jnp.int32)
            cged = plsc.load_gather(cgeb, [dvec])[0]
            totd = plsc.load_gather(totb, [dvec])[0]
            krem = krem - (cged - totd)
            prefix = prefix | lax.shift_left(d, jnp.int32(shift))
            d_last = d

        m_thr = prefix ^ min32
        iota16 = lax.iota(jnp.int32, l)
        eq_col = plsc.load_gather(hl, [iota16, jnp.full((l,), d_last,
                                                        jnp.int32)])
        prefix_eq = jnp.sum(jnp.where(iota16 < wid, eq_col, zeros))
        r_t = krem - prefix_eq

        run = jnp.int32(0)
        cntv = zeros
        for w in range(nw):
            tcps[w].wait()
            if w + 1 < nw:
                tcps[w + 1] = pltpu.async_copy(
                    tgt_hbm.at[pl.ds(base + (w + 1) * tw, tw)],
                    tbuf.at[(w + 1) % 2],
                    sem_t0 if (w + 1) % 2 == 0 else sem_t1)

            def fin_body(j, carry, w=w):
                run, cntv = carry
                ub = xbuf[pl.ds((w * twc + j) * l, l)]
                m = ub ^ min32
                tv = tbuf[w % 2, pl.ds(j * l, l)]
                pos = tv > 0.0
                gt = m > m_thr
                eq = m == m_thr
                eqi = jnp.where(eq, ones, zeros)
                rank = plsc.cumsum(eqi) + run
                sel = (gt | (eq & (rank <= r_t))) & pos
                cntv = cntv + jnp.where(sel, ones, zeros)
                run = run + plsc.all_reduce_population_count(eq)[0]
                return (run, cntv)

            run, cntv = plsc.parallel_loop(0, twc, unroll=8,
                                           carry=(run, cntv))(fin_body)

        hist[pl.ds(0, l)] = cntv
        pltpu.sync_copy(hist.at[pl.ds(0, l)], shc.at[wid])
        plsc.subcore_barrier()

        @pl.when(wid == 0)
        def _():
            pltpu.sync_copy(shc, cb)
            acc = cb[0, pl.ds(0, l)]
            for s in range(1, nt):
                acc = acc + cb[s, pl.ds(0, l)]
            num = jnp.sum(acc)
            val = -(num.astype(jnp.float32) * jnp.float32(1.0 / k))
            tbuf[0, pl.ds(0, l)] = jnp.full((l,), val, jnp.float32)
            pltpu.sync_copy(tbuf.at[0, pl.ds(0, l)], out_hbm)

    return pl.kernel(
        body,
        out_type=jax.ShapeDtypeStruct((l,), jnp.float32),
        mesh=mesh,
        compiler_params=pltpu.CompilerParams(
            use_tc_tiling_on_sc=False, needs_layout_passes=False
        ),
        scratch_types=[
            pltpu.VMEM((c,), jnp.int32),
            pltpu.VMEM((2, tw), jnp.float32),
            pltpu.VMEM((nb,), jnp.int32),
            pltpu.VMEM((nb, l), jnp.int32),
            pltpu.VMEM((nt, nb), jnp.int32),
            pltpu.VMEM((nb,), jnp.int32),
            pltpu.VMEM((nb,), jnp.int32),
            pltpu.VMEM((nt, l), jnp.int32),
            pltpu.VMEM((cap + 2 * l, ), jnp.int32),
            pltpu.VMEM_SHARED((nr, nt, nb), jnp.int32),
            pltpu.VMEM_SHARED((nt, l), jnp.int32),
            pltpu.SemaphoreType.DMA,
            pltpu.SemaphoreType.DMA,
            pltpu.SemaphoreType.DMA,
            pltpu.SemaphoreType.DMA,
        ],
    )


def kernel(predictions, targets):
    p = lax.bitcast_convert_type(predictions.reshape(-1), jnp.int32)
    t = targets.reshape(-1)
    out = _make(_N, _K)(p, t)
    return out[0]


# speedup vs baseline: 1.1005x; 1.1005x over previous
import jax
import jax.numpy as jnp
from jax import lax
from jax.experimental import pallas as pl
from jax.experimental.pallas import tpu as pltpu
from jax.experimental.pallas import tpu_sc as plsc

_N = 1048576
_K = max(1, int(_N * 0.2))
_NT = 16
_L = 16
_NB = 256
_NR = 4
_SHIFTS = (24, 16, 8, 0)
_MASKHI = (0, -(1 << 24), -(1 << 16), -(1 << 8))
_MIN32 = -(2**31)


def _make(n, k):
    nt, l, nb, nr = _NT, _L, _NB, _NR
    c = n // nt
    nchunk = c // l
    tw = min(c, 8192)
    nw = c // tw
    twc = tw // l
    cap = min(c, 32768)

    mesh = plsc.VectorSubcoreMesh(
        core_axis_name="c", subcore_axis_name="s", num_cores=1, num_subcores=nt
    )

    def body(pred_hbm, tgt_hbm, out_hbm, xbuf, tbuf, hist, hist2, hl,
             totb, cgeb, cb, cbuf, shh, shc, sem_a, sem_b, sem_t0, sem_t1):
        wid = lax.axis_index("s")
        base = wid * c
        half = c // 2
        cp0 = pltpu.async_copy(pred_hbm.at[pl.ds(base, half)],
                               xbuf.at[pl.ds(0, half)], sem_a)
        cp1 = pltpu.async_copy(pred_hbm.at[pl.ds(base + half, half)],
                               xbuf.at[pl.ds(half, half)], sem_b)
        tcps = [None] * nw
        tcps[0] = pltpu.async_copy(tgt_hbm.at[pl.ds(base, tw)],
                                   tbuf.at[0], sem_t0)

        ones = jnp.full((l,), 1, jnp.int32)
        zeros = jnp.full((l,), 0, jnp.int32)
        min32 = jnp.int32(_MIN32)
        lane = lax.iota(jnp.int32, l)

        prefix = jnp.int32(0)
        krem = jnp.int32(k)
        d_last = jnp.int32(0)

        for r in range(nr):
            shift = _SHIFTS[r]
            maskhi = jnp.int32(_MASKHI[r])
            shift_v = jnp.full((l,), shift, jnp.int32)

            def zero_body(j):
                for s in range(l):
                    hist2[j * l + s, pl.ds(0, l)] = zeros

            plsc.parallel_loop(0, nb // l, unroll=2)(zero_body)

            prefix_c = prefix

            def hist_body(i, carry, maskhi=maskhi, shift_v=shift_v,
                          prefix_c=prefix_c):
                cnt = carry
                sl = pl.ds(i * l, l)
                ub = xbuf[sl]
                match = (ub & maskhi) == prefix_c
                digit = lax.shift_right_logical(ub, shift_v) & 0xFF
                plsc.addupdate_scatter(hist2, [digit, lane], ones,
                                       mask=match)
                safe = jnp.minimum(cnt, jnp.int32(cap))
                plsc.store_compressed(cbuf.at[pl.ds(safe, l)], ub,
                                      mask=match)
                return cnt + plsc.all_reduce_population_count(match)[0]

            if r == 0:
                def r0_body(i, maskhi=maskhi, shift_v=shift_v,
                            prefix_c=prefix_c):
                    sl = pl.ds(i * l, l)
                    b = xbuf[sl]
                    ub = jnp.where(b >= 0, b ^ min32, ~b)
                    xbuf[sl] = ub
                    match = (ub & maskhi) == prefix_c
                    digit = lax.shift_right_logical(ub, shift_v) & 0xFF
                    plsc.addupdate_scatter(hist2, [digit, lane], ones,
                                           mask=match)

                cp0.wait()
                plsc.parallel_loop(0, nchunk // 2, unroll=8)(r0_body)
                cp1.wait()
                plsc.parallel_loop(nchunk // 2, nchunk, unroll=8)(r0_body)
            elif r == 1:
                cnt = plsc.parallel_loop(0, nchunk, unroll=4,
                                         carry=jnp.int32(0))(hist_body)
                lcnt = jnp.minimum(cnt, jnp.int32(cap))
                ovf = cnt > jnp.int32(cap)
                cb[0, pl.ds(0, l)] = jnp.where(ovf, ones, zeros)
                pltpu.sync_copy(cb.at[0], shc.at[wid])
            else:
                def compact_scan(maskhi=maskhi, shift_v=shift_v,
                                 prefix_c=prefix_c, lcnt=lcnt):
                    def cbody(i, _):
                        for u in range(2):
                            off = (i * 2 + u) * l
                            ub = cbuf[pl.ds(off, l)]
                            valid = (off + lane) < lcnt
                            match = valid & ((ub & maskhi) == prefix_c)
                            digit = (lax.shift_right_logical(ub, shift_v)
                                     & 0xFF)
                            plsc.addupdate_scatter(hist2, [digit, lane],
                                                   ones, mask=match)
                        return 0

                    lax.fori_loop(0, (lcnt + 2 * l - 1) // (2 * l), cbody, 0)

                def full_scan(maskhi=maskhi, shift_v=shift_v,
                              prefix_c=prefix_c):
                    def fbody(i, maskhi=maskhi, shift_v=shift_v,
                              prefix_c=prefix_c):
                        ub = xbuf[pl.ds(i * l, l)]
                        match = (ub & maskhi) == prefix_c
                        digit = lax.shift_right_logical(ub, shift_v) & 0xFF
                        plsc.addupdate_scatter(hist2, [digit, lane], ones,
                                               mask=match)

                    plsc.parallel_loop(0, nchunk, unroll=4)(fbody)

                lax.cond(use_compact, compact_scan, full_scan)

            def fold_body(j):
                rows = j * l + lane
                acc = zeros
                for s in range(l):
                    cols = (lane + s) & (l - 1)
                    acc = acc + plsc.load_gather(hist2, [rows, cols])
                hist[pl.ds(j * l, l)] = acc

            plsc.parallel_loop(0, nb // l, unroll=2)(fold_body)

            pltpu.sync_copy(hist, shh.at[r, wid])
            plsc.subcore_barrier()
            pltpu.sync_copy(shh.at[r], hl)

            if r == 1:
                pltpu.sync_copy(shc, cb)
                oacc = cb[0, pl.ds(0, l)]
                for s in range(1, nt):
                    oacc = oacc + cb[s, pl.ds(0, l)]
                use_compact = jnp.sum(oacc) == 0

            def red_body(cc):
                sl = pl.ds(cc * l, l)
                acc = hl[0, sl]
                for s in range(1, nt):
                    acc = acc + hl[s, sl]
                totb[sl] = acc

            plsc.parallel_loop(0, nb // l, unroll=2)(red_body)

            krem_c = krem

            def scan_body(j, carry, krem_c=krem_c):
                acc, nfound = carry
                cc = (nb // l - 1) - j
                sl = pl.ds(cc * l, l)
                t16 = totb[sl]
                csum = plsc.cumsum(t16)
                ctot = jnp.sum(t16)
                cge = (acc + ctot) - csum + t16
                cgeb[sl] = cge
                nfound = nfound + jnp.sum(jnp.where(cge >= krem_c, ones, zeros))
                return (acc + ctot, nfound)

            _, nfound = lax.fori_loop(
                0, nb // l, scan_body, (jnp.int32(0), jnp.int32(0))
            )
            d = nfound - 1
            dvec = jnp.full((l,), d, jnp.int32)
            cged = plsc.load_gather(cgeb, [dvec])[0]
            totd = plsc.load_gather(totb, [dvec])[0]
            krem = krem - (cged - totd)
            prefix = prefix | lax.shift_left(d, jnp.int32(shift))
            d_last = d

        m_thr = prefix ^ min32
        iota16 = lax.iota(jnp.int32, l)
        eq_col = plsc.load_gather(hl, [iota16, jnp.full((l,), d_last,
                                                        jnp.int32)])
        prefix_eq = jnp.sum(jnp.where(iota16 < wid, eq_col, zeros))
        r_t = krem - prefix_eq

        run = jnp.int32(0)
        cntv = zeros
        for w in range(nw):
            tcps[w].wait()
            if w + 1 < nw:
                tcps[w + 1] = pltpu.async_copy(
                    tgt_hbm.at[pl.ds(base + (w + 1) * tw, tw)],
                    tbuf.at[(w + 1) % 2],
                    sem_t0 if (w + 1) % 2 == 0 else sem_t1)

            def fin_body(j, carry, w=w):
                run, cntv = carry
                ub = xbuf[pl.ds((w * twc + j) * l, l)]
                m = ub ^ min32
                tv = tbuf[w % 2, pl.ds(j * l, l)]
                pos = tv > 0.0
                gt = m > m_thr
                eq = m == m_thr
                eqi = jnp.where(eq, ones, zeros)
                rank = plsc.cumsum(eqi) + run
                sel = (gt | (eq & (rank <= r_t))) & pos
                cntv = cntv + jnp.where(sel, ones, zeros)
                run = run + plsc.all_reduce_population_count(eq)[0]
                return (run, cntv)

            run, cntv = plsc.parallel_loop(0, twc, unroll=4,
                                           carry=(run, cntv))(fin_body)

        hist[pl.ds(0, l)] = cntv
        pltpu.sync_copy(hist.at[pl.ds(0, l)], shc.at[wid])
        plsc.subcore_barrier()

        @pl.when(wid == 0)
        def _():
            pltpu.sync_copy(shc, cb)
            acc = cb[0, pl.ds(0, l)]
            for s in range(1, nt):
                acc = acc + cb[s, pl.ds(0, l)]
            num = jnp.sum(acc)
            val = -(num.astype(jnp.float32) * jnp.float32(1.0 / k))
            tbuf[0, pl.ds(0, l)] = jnp.full((l,), val, jnp.float32)
            pltpu.sync_copy(tbuf.at[0, pl.ds(0, l)], out_hbm)

    return pl.kernel(
        body,
        out_type=jax.ShapeDtypeStruct((l,), jnp.float32),
        mesh=mesh,
        compiler_params=pltpu.CompilerParams(
            use_tc_tiling_on_sc=False, needs_layout_passes=False
        ),
        scratch_types=[
            pltpu.VMEM((c,), jnp.int32),
            pltpu.VMEM((2, tw), jnp.float32),
            pltpu.VMEM((nb,), jnp.int32),
            pltpu.VMEM((nb, l), jnp.int32),
            pltpu.VMEM((nt, nb), jnp.int32),
            pltpu.VMEM((nb,), jnp.int32),
            pltpu.VMEM((nb,), jnp.int32),
            pltpu.VMEM((nt, l), jnp.int32),
            pltpu.VMEM((cap + 2 * l, ), jnp.int32),
            pltpu.VMEM_SHARED((nr, nt, nb), jnp.int32),
            pltpu.VMEM_SHARED((nt, l), jnp.int32),
            pltpu.SemaphoreType.DMA,
            pltpu.SemaphoreType.DMA,
            pltpu.SemaphoreType.DMA,
            pltpu.SemaphoreType.DMA,
        ],
    )


def kernel(predictions, targets):
    p = lax.bitcast_convert_type(predictions.reshape(-1), jnp.int32)
    t = targets.reshape(-1)
    out = _make(_N, _K)(p, t)
    return out[0]


# speedup vs baseline: 1.1008x; 1.0003x over previous
import jax
import jax.numpy as jnp
from jax import lax
from jax.experimental import pallas as pl
from jax.experimental.pallas import tpu as pltpu
from jax.experimental.pallas import tpu_sc as plsc

_N = 1048576
_K = max(1, int(_N * 0.2))
_NT = 16
_L = 16
_NB = 256
_NR = 4
_SHIFTS = (24, 16, 8, 0)
_MASKHI = (0, -(1 << 24), -(1 << 16), -(1 << 8))
_MIN32 = -(2**31)


def _make(n, k):
    nt, l, nb, nr = _NT, _L, _NB, _NR
    c = n // nt
    nchunk = c // l
    tw = min(c, 8192)
    nw = c // tw
    twc = tw // l
    cap = min(c, 32768)

    mesh = plsc.VectorSubcoreMesh(
        core_axis_name="c", subcore_axis_name="s", num_cores=1, num_subcores=nt
    )

    def body(pred_hbm, tgt_hbm, out_hbm, xbuf, tbuf, hist, hist2, hl,
             totb, cgeb, cb, cbuf, shh, shc, sem_a, sem_b, sem_t0, sem_t1):
        wid = lax.axis_index("s")
        base = wid * c
        half = c // 2
        cp0 = pltpu.async_copy(pred_hbm.at[pl.ds(base, half)],
                               xbuf.at[pl.ds(0, half)], sem_a)
        cp1 = pltpu.async_copy(pred_hbm.at[pl.ds(base + half, half)],
                               xbuf.at[pl.ds(half, half)], sem_b)
        tcps = [None] * nw
        tcps[0] = pltpu.async_copy(tgt_hbm.at[pl.ds(base, tw)],
                                   tbuf.at[0], sem_t0)

        ones = jnp.full((l,), 1, jnp.int32)
        zeros = jnp.full((l,), 0, jnp.int32)
        min32 = jnp.int32(_MIN32)
        lane = lax.iota(jnp.int32, l)

        prefix = jnp.int32(0)
        krem = jnp.int32(k)
        d_last = jnp.int32(0)

        for r in range(nr):
            shift = _SHIFTS[r]
            maskhi = jnp.int32(_MASKHI[r])
            shift_v = jnp.full((l,), shift, jnp.int32)

            def zero_body(j):
                for s in range(l):
                    hist2[j * l + s, pl.ds(0, l)] = zeros

            plsc.parallel_loop(0, nb // l, unroll=2)(zero_body)

            prefix_c = prefix

            def hist_body(i, carry, maskhi=maskhi, shift_v=shift_v,
                          prefix_c=prefix_c):
                cnt = carry
                sl = pl.ds(i * l, l)
                ub = xbuf[sl]
                match = (ub & maskhi) == prefix_c
                digit = lax.shift_right_logical(ub, shift_v) & 0xFF
                plsc.addupdate_scatter(hist2, [digit, lane], ones,
                                       mask=match)
                safe = jnp.minimum(cnt, jnp.int32(cap))
                plsc.store_compressed(cbuf.at[pl.ds(safe, l)], ub,
                                      mask=match)
                return cnt + plsc.all_reduce_population_count(match)[0]

            if r == 0:
                def r0_body(i, maskhi=maskhi, shift_v=shift_v,
                            prefix_c=prefix_c):
                    sl = pl.ds(i * l, l)
                    b = xbuf[sl]
                    ub = jnp.where(b >= 0, b ^ min32, ~b)
                    xbuf[sl] = ub
                    match = (ub & maskhi) == prefix_c
                    digit = lax.shift_right_logical(ub, shift_v) & 0xFF
                    plsc.addupdate_scatter(hist2, [digit, lane], ones,
                                           mask=match)

                cp0.wait()
                plsc.parallel_loop(0, nchunk // 2, unroll=4)(r0_body)
                cp1.wait()
                plsc.parallel_loop(nchunk // 2, nchunk, unroll=4)(r0_body)
            elif r == 1:
                cnt = plsc.parallel_loop(0, nchunk, unroll=4,
                                         carry=jnp.int32(0))(hist_body)
                lcnt = jnp.minimum(cnt, jnp.int32(cap))
                ovf = cnt > jnp.int32(cap)
                cb[0, pl.ds(0, l)] = jnp.where(ovf, ones, zeros)
                pltpu.sync_copy(cb.at[0], shc.at[wid])
            else:
                def compact_scan(maskhi=maskhi, shift_v=shift_v,
                                 prefix_c=prefix_c, lcnt=lcnt):
                    def cbody(i, _):
                        for u in range(2):
                            off = (i * 2 + u) * l
                            ub = cbuf[pl.ds(off, l)]
                            valid = (off + lane) < lcnt
                            match = valid & ((ub & maskhi) == prefix_c)
                            digit = (lax.shift_right_logical(ub, shift_v)
                                     & 0xFF)
                            plsc.addupdate_scatter(hist2, [digit, lane],
                                                   ones, mask=match)
                        return 0

                    lax.fori_loop(0, (lcnt + 2 * l - 1) // (2 * l), cbody, 0)

                def full_scan(maskhi=maskhi, shift_v=shift_v,
                              prefix_c=prefix_c):
                    def fbody(i, maskhi=maskhi, shift_v=shift_v,
                              prefix_c=prefix_c):
                        ub = xbuf[pl.ds(i * l, l)]
                        match = (ub & maskhi) == prefix_c
                        digit = lax.shift_right_logical(ub, shift_v) & 0xFF
                        plsc.addupdate_scatter(hist2, [digit, lane], ones,
                                               mask=match)

                    plsc.parallel_loop(0, nchunk, unroll=4)(fbody)

                lax.cond(use_compact, compact_scan, full_scan)

            def fold_body(j):
                rows = j * l + lane
                acc = zeros
                for s in range(l):
                    cols = (lane + s) & (l - 1)
                    acc = acc + plsc.load_gather(hist2, [rows, cols])
                hist[pl.ds(j * l, l)] = acc

            plsc.parallel_loop(0, nb // l, unroll=2)(fold_body)

            pltpu.sync_copy(hist, shh.at[r, wid])
            plsc.subcore_barrier()
            pltpu.sync_copy(shh.at[r], hl)

            if r == 1:
                pltpu.sync_copy(shc, cb)
                oacc = cb[0, pl.ds(0, l)]
                for s in range(1, nt):
                    oacc = oacc + cb[s, pl.ds(0, l)]
                use_compact = jnp.sum(oacc) == 0

            def red_body(cc):
                sl = pl.ds(cc * l, l)
                acc = hl[0, sl]
                for s in range(1, nt):
                    acc = acc + hl[s, sl]
                totb[sl] = acc

            plsc.parallel_loop(0, nb // l, unroll=2)(red_body)

            krem_c = krem

            def scan_body(j, carry, krem_c=krem_c):
                acc, nfound = carry
                cc = (nb // l - 1) - j
                sl = pl.ds(cc * l, l)
                t16 = totb[sl]
                csum = plsc.cumsum(t16)
                ctot = jnp.sum(t16)
                cge = (acc + ctot) - csum + t16
                cgeb[sl] = cge
                nfound = nfound + jnp.sum(jnp.where(cge >= krem_c, ones, zeros))
                return (acc + ctot, nfound)

            _, nfound = lax.fori_loop(
                0, nb // l, scan_body, (jnp.int32(0), jnp.int32(0))
            )
            d = nfound - 1
            dvec = jnp.full((l,), d, jnp.int32)
            cged = plsc.load_gather(cgeb, [dvec])[0]
            totd = plsc.load_gather(totb, [dvec])[0]
            krem = krem - (cged - totd)
            prefix = prefix | lax.shift_left(d, jnp.int32(shift))
            d_last = d

        m_thr = prefix ^ min32
        iota16 = lax.iota(jnp.int32, l)
        eq_col = plsc.load_gather(hl, [iota16, jnp.full((l,), d_last,
                                                        jnp.int32)])
        prefix_eq = jnp.sum(jnp.where(iota16 < wid, eq_col, zeros))
        r_t = krem - prefix_eq

        run = jnp.int32(0)
        cntv = zeros
        for w in range(nw):
            tcps[w].wait()
            if w + 1 < nw:
                tcps[w + 1] = pltpu.async_copy(
                    tgt_hbm.at[pl.ds(base + (w + 1) * tw, tw)],
                    tbuf.at[(w + 1) % 2],
                    sem_t0 if (w + 1) % 2 == 0 else sem_t1)

            def fin_body(j, carry, w=w):
                run, cntv = carry
                ub = xbuf[pl.ds((w * twc + j) * l, l)]
                m = ub ^ min32
                tv = tbuf[w % 2, pl.ds(j * l, l)]
                pos = tv > 0.0
                gt = m > m_thr
                eq = m == m_thr
                eqi = jnp.where(eq, ones, zeros)
                rank = plsc.cumsum(eqi) + run
                sel = (gt | (eq & (rank <= r_t))) & pos
                cntv = cntv + jnp.where(sel, ones, zeros)
                run = run + plsc.all_reduce_population_count(eq)[0]
                return (run, cntv)

            run, cntv = plsc.parallel_loop(0, twc, unroll=4,
                                           carry=(run, cntv))(fin_body)

        hist[pl.ds(0, l)] = cntv
        pltpu.sync_copy(hist.at[pl.ds(0, l)], shc.at[wid])
        plsc.subcore_barrier()

        @pl.when(wid == 0)
        def _():
            pltpu.sync_copy(shc, cb)
            acc = cb[0, pl.ds(0, l)]
            for s in range(1, nt):
                acc = acc + cb[s, pl.ds(0, l)]
            num = jnp.sum(acc)
            val = -(num.astype(jnp.float32) * jnp.float32(1.0 / k))
            tbuf[0, pl.ds(0, l)] = jnp.full((l,), val, jnp.float32)
            pltpu.sync_copy(tbuf.at[0, pl.ds(0, l)], out_hbm)

    return pl.kernel(
        body,
        out_type=jax.ShapeDtypeStruct((l,), jnp.float32),
        mesh=mesh,
        compiler_params=pltpu.CompilerParams(
            use_tc_tiling_on_sc=False, needs_layout_passes=False
        ),
        scratch_types=[
            pltpu.VMEM((c,), jnp.int32),
            pltpu.VMEM((2, tw), jnp.float32),
            pltpu.VMEM((nb,), jnp.int32),
            pltpu.VMEM((nb, l), jnp.int32),
            pltpu.VMEM((nt, nb), jnp.int32),
            pltpu.VMEM((nb,), jnp.int32),
            pltpu.VMEM((nb,), jnp.int32),
            pltpu.VMEM((nt, l), jnp.int32),
            pltpu.VMEM((cap + 2 * l, ), jnp.int32),
            pltpu.VMEM_SHARED((nr, nt, nb), jnp.int32),
            pltpu.VMEM_SHARED((nt, l), jnp.int32),
            pltpu.SemaphoreType.DMA,
            pltpu.SemaphoreType.DMA,
            pltpu.SemaphoreType.DMA,
            pltpu.SemaphoreType.DMA,
        ],
    )


def kernel(predictions, targets):
    p = lax.bitcast_convert_type(predictions.reshape(-1), jnp.int32)
    t = targets.reshape(-1)
    out = _make(_N, _K)(p, t)
    return out[0]
